# Initial kernel scaffold; baseline (speedup 1.0000x reference)
import jax
import jax.numpy as jnp
from jax.experimental import pallas as pl


def kernel(hidden_states, codebook_C, mapping_M):
    raise NotImplementedError("write your pallas kernel here")


# speedup vs baseline: 2.9490x; 2.9490x over previous
import functools

import jax
import jax.numpy as jnp
from jax import lax
from jax.experimental import pallas as pl
from jax.experimental.pallas import tpu as pltpu
from jax.experimental.pallas import tpu_sc as plsc

B, S, H = 8, 16, 4096
K = 8192
V = 100000
R = B * S


_N_BLK = 1024


def _matmul_body(hs_ref, c_ref, out_ref):
    out_ref[...] = lax.dot_general(
        hs_ref[...], c_ref[...],
        dimension_numbers=(((1,), (1,)), ((), ())),
        preferred_element_type=jnp.float32,
    )


def _codebook_logits(hs2d, codebook):
    grid = (K // _N_BLK,)
    return pl.pallas_call(
        _matmul_body,
        grid=grid,
        in_specs=[
            pl.BlockSpec((R, H), lambda i: (0, 0)),
            pl.BlockSpec((_N_BLK, H), lambda i: (i, 0)),
        ],
        out_specs=pl.BlockSpec((R, _N_BLK), lambda i: (0, i)),
        out_shape=jax.ShapeDtypeStruct((R, K), jnp.float32),
    )(hs2d, codebook)


_NW = 32
_RPW = R // _NW
_VCH = 4000
_NCH = V // _VCH


def _sc_expand_body(lc_hbm, m_hbm, out_hbm, rows_v, idx_v, buf_v):
    nc = 2
    wid = lax.axis_index("s") * nc + lax.axis_index("c")
    row_base = pl.multiple_of(wid * (_RPW * K), 8)
    pltpu.sync_copy(lc_hbm.at[pl.ds(row_base, _RPW * K)], rows_v)

    def chunk_body(ci, _):
        v0 = pl.multiple_of(ci * _VCH, 8)
        pltpu.sync_copy(m_hbm.at[pl.ds(v0, _VCH)], idx_v)

        def vec_body(j, _):
            off = pl.multiple_of(j * 16, 16)
            idx16 = idx_v[pl.ds(off, 16)]
            for r in range(_RPW):
                flat_idx = idx16 + jnp.full((16,), r * K, jnp.int32)
                vals = plsc.load_gather(rows_v, [flat_idx])
                buf_v[pl.ds(r * _VCH + off, 16)] = vals
            return ()

        lax.fori_loop(0, _VCH // 16, vec_body, (), unroll=4)
        for r in range(_RPW):
            dst0 = pl.multiple_of((wid * _RPW + r) * V + v0, 8)
            pltpu.sync_copy(buf_v.at[pl.ds(r * _VCH, _VCH)],
                            out_hbm.at[pl.ds(dst0, _VCH)])
        return ()

    lax.fori_loop(0, _NCH, chunk_body, ())


def _expand_vocab(lc_flat, mapping):
    f = functools.partial(
        pl.kernel,
        out_type=jax.ShapeDtypeStruct((R * V,), jnp.float32),
        mesh=plsc.VectorSubcoreMesh(core_axis_name="c", subcore_axis_name="s"),
        compiler_params=pltpu.CompilerParams(use_tc_tiling_on_sc=False,
                                             needs_layout_passes=False),
        scratch_types=[
            pltpu.VMEM((_RPW * K,), jnp.float32),
            pltpu.VMEM((_VCH,), jnp.int32),
            pltpu.VMEM((_RPW * _VCH,), jnp.float32),
        ],
    )(_sc_expand_body)
    return f(lc_flat, mapping)


def kernel(hidden_states, codebook_C, mapping_M):
    hs2d = hidden_states.reshape(R, H)
    lc = _codebook_logits(hs2d, codebook_C)
    m32 = mapping_M.astype(jnp.int32)
    out_flat = _expand_vocab(lc.reshape(R * K), m32)
    return out_flat.reshape(B, S, V)


# speedup vs baseline: 3.4639x; 1.1746x over previous
import functools

import jax
import jax.numpy as jnp
from jax import lax
from jax.experimental import pallas as pl
from jax.experimental.pallas import tpu as pltpu
from jax.experimental.pallas import tpu_sc as plsc

B, S, H = 8, 16, 4096
K = 8192
V = 100000
R = B * S


_N_BLK = 1024


def _matmul_body(hs_ref, c_ref, out_ref):
    out_ref[...] = lax.dot_general(
        hs_ref[...], c_ref[...],
        dimension_numbers=(((1,), (1,)), ((), ())),
        preferred_element_type=jnp.float32,
    )


def _codebook_logits(hs2d, codebook):
    grid = (K // _N_BLK,)
    return pl.pallas_call(
        _matmul_body,
        grid=grid,
        in_specs=[
            pl.BlockSpec((R, H), lambda i: (0, 0)),
            pl.BlockSpec((_N_BLK, H), lambda i: (i, 0)),
        ],
        out_specs=pl.BlockSpec((R, _N_BLK), lambda i: (0, i)),
        out_shape=jax.ShapeDtypeStruct((R, K), jnp.float32),
    )(hs2d, codebook)


_NW = 32
_RPW = R // _NW
_VCH = 2000
_NCH = V // _VCH


def _sc_expand_body(lc_hbm, m_hbm, out_hbm, rows_v, idx_v, buf_v,
                    sem_i0, sem_i1, sem_o0, sem_o1):
    nc = 2
    wid = lax.axis_index("s") * nc + lax.axis_index("c")
    sem_i = (sem_i0, sem_i1)
    sem_o = (sem_o0, sem_o1)
    row_base = pl.multiple_of(wid * (_RPW * K), 8)
    pltpu.sync_copy(lc_hbm.at[pl.ds(row_base, _RPW * K)], rows_v)

    def idx_copy(ci, b):
        v0 = pl.multiple_of(ci * _VCH, 8)
        return pltpu.make_async_copy(m_hbm.at[pl.ds(v0, _VCH)],
                                     idx_v.at[b], sem_i[b])

    def out_copy(ci, b, r):
        v0 = (wid * _RPW + r) * V + ci * _VCH
        v0 = pl.multiple_of(v0, 8)
        return pltpu.make_async_copy(buf_v.at[b, r],
                                     out_hbm.at[pl.ds(v0, _VCH)], sem_o[b])

    idx_copy(0, 0).start()
    idx_copy(1, 1).start()

    def outer(i, _):
        g = pl.multiple_of(i * 2, 2)
        for b in range(2):
            ci = g + b
            @pl.when(ci >= 2)
            def _():
                for r in range(_RPW):
                    out_copy(ci, b, r).wait()
            idx_copy(ci, b).wait()

            def vec_body(j, _):
                off = pl.multiple_of(j * 16, 16)
                idx16 = idx_v[b, pl.ds(off, 16)]
                for r in range(_RPW):
                    flat_idx = idx16 + jnp.full((16,), r * K, jnp.int32)
                    vals = plsc.load_gather(rows_v, [flat_idx])
                    buf_v[b, r, pl.ds(off, 16)] = vals
                return ()

            lax.fori_loop(0, _VCH // 16, vec_body, (), unroll=8)
            for r in range(_RPW):
                out_copy(ci, b, r).start()

            @pl.when(ci + 2 < _NCH)
            def _():
                idx_copy(ci + 2, b).start()
        return ()

    lax.fori_loop(0, _NCH // 2, outer, ())
    for b in range(2):
        for r in range(_RPW):
            out_copy(_NCH - 2 + b, b, r).wait()


def _expand_vocab(lc_flat, mapping):
    f = functools.partial(
        pl.kernel,
        out_type=jax.ShapeDtypeStruct((R * V,), jnp.float32),
        mesh=plsc.VectorSubcoreMesh(core_axis_name="c", subcore_axis_name="s"),
        compiler_params=pltpu.CompilerParams(use_tc_tiling_on_sc=False,
                                             needs_layout_passes=False),
        scratch_types=[
            pltpu.VMEM((_RPW * K,), jnp.float32),
            pltpu.VMEM((2, _VCH), jnp.int32),
            pltpu.VMEM((2, _RPW, _VCH), jnp.float32),
            pltpu.SemaphoreType.DMA,
            pltpu.SemaphoreType.DMA,
            pltpu.SemaphoreType.DMA,
            pltpu.SemaphoreType.DMA,
        ],
    )(_sc_expand_body)
    return f(lc_flat, mapping)


def kernel(hidden_states, codebook_C, mapping_M):
    hs2d = hidden_states.reshape(R, H)
    lc = _codebook_logits(hs2d, codebook_C)
    m32 = mapping_M.astype(jnp.int32)
    out_flat = _expand_vocab(lc.reshape(R * K), m32)
    return out_flat.reshape(B, S, V)


# speedup vs baseline: 4.7434x; 1.3694x over previous
import functools

import jax
import jax.numpy as jnp
from jax import lax
from jax.experimental import pallas as pl
from jax.experimental.pallas import tpu as pltpu
from jax.experimental.pallas import tpu_sc as plsc

B, S, H = 8, 16, 4096
K = 8192
V = 100000
R = B * S


_N_BLK = 1024


def _matmul_body(hs_ref, c_ref, out_ref):
    out_ref[...] = lax.dot_general(
        hs_ref[...], c_ref[...],
        dimension_numbers=(((1,), (1,)), ((), ())),
        preferred_element_type=jnp.float32,
    )


def _codebook_logits(hs2d, codebook):
    grid = (K // _N_BLK,)
    return pl.pallas_call(
        _matmul_body,
        grid=grid,
        in_specs=[
            pl.BlockSpec((R, H), lambda i: (0, 0)),
            pl.BlockSpec((_N_BLK, H), lambda i: (i, 0)),
        ],
        out_specs=pl.BlockSpec((R, _N_BLK), lambda i: (0, i)),
        out_shape=jax.ShapeDtypeStruct((R, K), jnp.float32),
    )(hs2d, codebook)


_NW = 32
_RPW = R // _NW
_VCH = 2000
_NCH = V // _VCH


def _sc_expand_body(lc_hbm, m_hbm, out_hbm, rows_v, idx_v, buf_v,
                    sem_i0, sem_i1, sem_o0, sem_o1):
    nc = 2
    wid = lax.axis_index("s") * nc + lax.axis_index("c")
    sem_i = (sem_i0, sem_i1)
    sem_o = (sem_o0, sem_o1)
    row_base = pl.multiple_of(wid * (_RPW * K), 8)
    pltpu.sync_copy(lc_hbm.at[pl.ds(row_base, _RPW * K)], rows_v)

    def idx_copy(ci, b):
        v0 = pl.multiple_of(ci * _VCH, 8)
        return pltpu.make_async_copy(m_hbm.at[pl.ds(v0, _VCH)],
                                     idx_v.at[b], sem_i[b])

    def out_copy(ci, b, r):
        v0 = (wid * _RPW + r) * V + ci * _VCH
        v0 = pl.multiple_of(v0, 8)
        return pltpu.make_async_copy(buf_v.at[b, r],
                                     out_hbm.at[pl.ds(v0, _VCH)], sem_o[b])

    idx_copy(0, 0).start()
    idx_copy(1, 1).start()

    def outer(i, _):
        g = pl.multiple_of(i * 2, 2)
        for b in range(2):
            ci = g + b
            @pl.when(ci >= 2)
            def _():
                for r in range(_RPW):
                    out_copy(ci, b, r).wait()
            idx_copy(ci, b).wait()

            U = 5

            def vec_body(j, _):
                base = pl.multiple_of(j * (16 * U), 16)
                offs = [base + u * 16 for u in range(U)]
                idxs = [idx_v[b, pl.ds(o, 16)] for o in offs]
                for r in range(_RPW):
                    row_ref = rows_v.at[pl.ds(r * K, K)]
                    vals = [plsc.load_gather(row_ref, [ix]) for ix in idxs]
                    for u in range(U):
                        buf_v[b, r, pl.ds(offs[u], 16)] = vals[u]
                return ()

            lax.fori_loop(0, _VCH // (16 * U), vec_body, (), unroll=2)
            for r in range(_RPW):
                out_copy(ci, b, r).start()

            @pl.when(ci + 2 < _NCH)
            def _():
                idx_copy(ci + 2, b).start()
        return ()

    lax.fori_loop(0, _NCH // 2, outer, ())
    for b in range(2):
        for r in range(_RPW):
            out_copy(_NCH - 2 + b, b, r).wait()


def _expand_vocab(lc_flat, mapping):
    f = functools.partial(
        pl.kernel,
        out_type=jax.ShapeDtypeStruct((R * V,), jnp.float32),
        mesh=plsc.VectorSubcoreMesh(core_axis_name="c", subcore_axis_name="s"),
        compiler_params=pltpu.CompilerParams(use_tc_tiling_on_sc=False,
                                             needs_layout_passes=False),
        scratch_types=[
            pltpu.VMEM((_RPW * K,), jnp.float32),
            pltpu.VMEM((2, _VCH), jnp.int32),
            pltpu.VMEM((2, _RPW, _VCH), jnp.float32),
            pltpu.SemaphoreType.DMA,
            pltpu.SemaphoreType.DMA,
            pltpu.SemaphoreType.DMA,
            pltpu.SemaphoreType.DMA,
        ],
    )(_sc_expand_body)
    return f(lc_flat, mapping)


def kernel(hidden_states, codebook_C, mapping_M):
    hs2d = hidden_states.reshape(R, H)
    lc = _codebook_logits(hs2d, codebook_C)
    m32 = mapping_M.astype(jnp.int32)
    out_flat = _expand_vocab(lc.reshape(R * K), m32)
    return out_flat.reshape(B, S, V)


# speedup vs baseline: 5.5373x; 1.1674x over previous
import functools

import jax
import jax.numpy as jnp
from jax import lax
from jax.experimental import pallas as pl
from jax.experimental.pallas import tpu as pltpu
from jax.experimental.pallas import tpu_sc as plsc

B, S, H = 8, 16, 4096
K = 8192
V = 100000
R = B * S


_N_BLK = 1024


def _matmul_body(hs_ref, c_ref, out_ref):
    out_ref[...] = lax.dot_general(
        hs_ref[...], c_ref[...],
        dimension_numbers=(((1,), (1,)), ((), ())),
        preferred_element_type=jnp.float32,
    )


def _codebook_logits(hs2d, codebook):
    grid = (K // _N_BLK,)
    return pl.pallas_call(
        _matmul_body,
        grid=grid,
        in_specs=[
            pl.BlockSpec((R, H), lambda i: (0, 0)),
            pl.BlockSpec((_N_BLK, H), lambda i: (i, 0)),
        ],
        out_specs=pl.BlockSpec((R, _N_BLK), lambda i: (0, i)),
        out_shape=jax.ShapeDtypeStruct((R, K), jnp.float32),
    )(hs2d, codebook)


_NW = 32
_RPW = R // _NW
_VCH = 2000
_NCH = V // _VCH
_VP = (V + 127) // 128 * 128


def _sc_expand_body(lc_hbm, m_hbm, out_hbm, rows_v, idx_v, buf_v,
                    sem_i0, sem_i1, sem_o0, sem_o1):
    nc = 2
    wid = lax.axis_index("s") * nc + lax.axis_index("c")
    sem_i = (sem_i0, sem_i1)
    sem_o = (sem_o0, sem_o1)
    row_base = pl.multiple_of(wid * (_RPW * K), 8)
    pltpu.sync_copy(lc_hbm.at[pl.ds(row_base, _RPW * K)], rows_v)

    def idx_copy(ci, b):
        v0 = pl.multiple_of(ci * _VCH, 8)
        return pltpu.make_async_copy(m_hbm.at[pl.ds(v0, _VCH)],
                                     idx_v.at[b], sem_i[b])

    def out_copy(ci, b, r):
        v0 = (wid * _RPW + r) * _VP + ci * _VCH
        v0 = pl.multiple_of(v0, 8)
        return pltpu.make_async_copy(buf_v.at[b, r],
                                     out_hbm.at[pl.ds(v0, _VCH)], sem_o[b])

    idx_copy(0, 0).start()
    idx_copy(1, 1).start()

    def outer(i, _):
        g = pl.multiple_of(i * 2, 2)
        for b in range(2):
            ci = g + b
            @pl.when(ci >= 2)
            def _():
                for r in range(_RPW):
                    out_copy(ci, b, r).wait()
            idx_copy(ci, b).wait()

            U = 5

            @plsc.parallel_loop(0, _VCH // (16 * U), 1, unroll=2)
            def _(j):
                base = pl.multiple_of(j * (16 * U), 16)
                offs = [base + u * 16 for u in range(U)]
                idxs = [idx_v[b, pl.ds(o, 16)] for o in offs]
                for r in range(_RPW):
                    row_ref = rows_v.at[pl.ds(r * K, K)]
                    vals = [plsc.load_gather(row_ref, [ix]) for ix in idxs]
                    for u in range(U):
                        buf_v[b, r, pl.ds(offs[u], 16)] = vals[u]
            for r in range(_RPW):
                out_copy(ci, b, r).start()

            @pl.when(ci + 2 < _NCH)
            def _():
                idx_copy(ci + 2, b).start()
        return ()

    lax.fori_loop(0, _NCH // 2, outer, ())
    for b in range(2):
        for r in range(_RPW):
            out_copy(_NCH - 2 + b, b, r).wait()


def _retile_body(flat_hbm, out_ref, sems):
    j = pl.program_id(0)
    for u in range(8):
        src = flat_hbm.at[pl.ds(pl.multiple_of((8 * j + u) * _VP, 128), _VP)]
        pltpu.make_async_copy(src, out_ref.at[0, u], sems.at[u]).start()
    for u in range(8):
        pltpu.make_async_copy(flat_hbm.at[pl.ds(0, _VP)],
                              out_ref.at[0, u], sems.at[u]).wait()


def _retile(flat):
    return pl.pallas_call(
        _retile_body,
        grid=(R // 8,),
        in_specs=[pl.BlockSpec(memory_space=pl.ANY)],
        out_specs=pl.BlockSpec((1, 8, _VP), lambda j: (j // 2, j % 2, 0)),
        out_shape=jax.ShapeDtypeStruct((B, S, V), jnp.float32),
        scratch_shapes=[pltpu.SemaphoreType.DMA((8,))],
    )(flat)


def _expand_vocab(lc_flat, mapping):
    f = functools.partial(
        pl.kernel,
        out_type=jax.ShapeDtypeStruct((R * _VP,), jnp.float32),
        mesh=plsc.VectorSubcoreMesh(core_axis_name="c", subcore_axis_name="s"),
        compiler_params=pltpu.CompilerParams(use_tc_tiling_on_sc=False,
                                             needs_layout_passes=False),
        scratch_types=[
            pltpu.VMEM((_RPW * K,), jnp.float32),
            pltpu.VMEM((2, _VCH), jnp.int32),
            pltpu.VMEM((2, _RPW, _VCH), jnp.float32),
            pltpu.SemaphoreType.DMA,
            pltpu.SemaphoreType.DMA,
            pltpu.SemaphoreType.DMA,
            pltpu.SemaphoreType.DMA,
        ],
    )(_sc_expand_body)
    return f(lc_flat, mapping)


def kernel(hidden_states, codebook_C, mapping_M):
    hs2d = hidden_states.reshape(R, H)
    lc = _codebook_logits(hs2d, codebook_C)
    m32 = mapping_M.astype(jnp.int32)
    out_flat = _expand_vocab(lc.reshape(R * K), m32)
    return _retile(out_flat)


# speedup vs baseline: 5.5598x; 1.0041x over previous
import functools

import jax
import jax.numpy as jnp
from jax import lax
from jax.experimental import pallas as pl
from jax.experimental.pallas import tpu as pltpu
from jax.experimental.pallas import tpu_sc as plsc

B, S, H = 8, 16, 4096
K = 8192
V = 100000
R = B * S


_N_BLK = 1024


def _matmul_body(hs_ref, c_ref, out_ref):
    out_ref[...] = lax.dot_general(
        hs_ref[...], c_ref[...],
        dimension_numbers=(((1,), (1,)), ((), ())),
        preferred_element_type=jnp.float32,
    )


def _codebook_logits(hs2d, codebook):
    grid = (K // _N_BLK,)
    return pl.pallas_call(
        _matmul_body,
        grid=grid,
        in_specs=[
            pl.BlockSpec((R, H), lambda i: (0, 0)),
            pl.BlockSpec((_N_BLK, H), lambda i: (i, 0)),
        ],
        out_specs=pl.BlockSpec((R, _N_BLK), lambda i: (0, i)),
        out_shape=jax.ShapeDtypeStruct((R, K), jnp.float32),
    )(hs2d, codebook)


_NW = 32
_RPW = R // _NW
_VCH = 2000
_NCH = V // _VCH
_VP = (V + 127) // 128 * 128


def _sc_expand_body(lc_hbm, m_hbm, out_hbm, rows_v, idx_v, buf_v,
                    sem_i0, sem_i1, sem_o0, sem_o1):
    nc = 2
    wid = lax.axis_index("s") * nc + lax.axis_index("c")
    sem_i = (sem_i0, sem_i1)
    sem_o = (sem_o0, sem_o1)
    row_base = pl.multiple_of(wid * (_RPW * K), 8)
    pltpu.sync_copy(lc_hbm.at[pl.ds(row_base, _RPW * K)], rows_v)

    def idx_copy(ci, b):
        v0 = pl.multiple_of(ci * _VCH, 8)
        return pltpu.make_async_copy(m_hbm.at[pl.ds(v0, _VCH)],
                                     idx_v.at[b], sem_i[b])

    def out_copy(ci, b, r):
        v0 = (wid * _RPW + r) * _VP + ci * _VCH
        v0 = pl.multiple_of(v0, 8)
        return pltpu.make_async_copy(buf_v.at[b, r],
                                     out_hbm.at[pl.ds(v0, _VCH)], sem_o[b])

    idx_copy(0, 0).start()
    idx_copy(1, 1).start()

    def outer(i, _):
        g = pl.multiple_of(i * 2, 2)
        for b in range(2):
            ci = g + b
            @pl.when(ci >= 2)
            def _():
                for r in range(_RPW):
                    out_copy(ci, b, r).wait()
            idx_copy(ci, b).wait()

            U = 5

            @plsc.parallel_loop(0, _VCH // (16 * U), 1, unroll=5)
            def _(j):
                base = pl.multiple_of(j * (16 * U), 16)
                offs = [base + u * 16 for u in range(U)]
                idxs = [idx_v[b, pl.ds(o, 16)] for o in offs]
                for r in range(_RPW):
                    row_ref = rows_v.at[pl.ds(r * K, K)]
                    vals = [plsc.load_gather(row_ref, [ix]) for ix in idxs]
                    for u in range(U):
                        buf_v[b, r, pl.ds(offs[u], 16)] = vals[u]
            for r in range(_RPW):
                out_copy(ci, b, r).start()

            @pl.when(ci + 2 < _NCH)
            def _():
                idx_copy(ci + 2, b).start()
        return ()

    lax.fori_loop(0, _NCH // 2, outer, ())
    for b in range(2):
        for r in range(_RPW):
            out_copy(_NCH - 2 + b, b, r).wait()


def _retile_body(flat_hbm, out_ref, sems):
    j = pl.program_id(0)
    for u in range(8):
        src = flat_hbm.at[pl.ds(pl.multiple_of((8 * j + u) * _VP, 128), _VP)]
        pltpu.make_async_copy(src, out_ref.at[0, u], sems.at[u]).start()
    for u in range(8):
        pltpu.make_async_copy(flat_hbm.at[pl.ds(0, _VP)],
                              out_ref.at[0, u], sems.at[u]).wait()


def _retile(flat):
    return pl.pallas_call(
        _retile_body,
        grid=(R // 8,),
        in_specs=[pl.BlockSpec(memory_space=pl.ANY)],
        out_specs=pl.BlockSpec((1, 8, _VP), lambda j: (j // 2, j % 2, 0)),
        out_shape=jax.ShapeDtypeStruct((B, S, V), jnp.float32),
        scratch_shapes=[pltpu.SemaphoreType.DMA((8,))],
    )(flat)


def _expand_vocab(lc_flat, mapping):
    f = functools.partial(
        pl.kernel,
        out_type=jax.ShapeDtypeStruct((R * _VP,), jnp.float32),
        mesh=plsc.VectorSubcoreMesh(core_axis_name="c", subcore_axis_name="s"),
        compiler_params=pltpu.CompilerParams(use_tc_tiling_on_sc=False,
                                             needs_layout_passes=False),
        scratch_types=[
            pltpu.VMEM((_RPW * K,), jnp.float32),
            pltpu.VMEM((2, _VCH), jnp.int32),
            pltpu.VMEM((2, _RPW, _VCH), jnp.float32),
            pltpu.SemaphoreType.DMA,
            pltpu.SemaphoreType.DMA,
            pltpu.SemaphoreType.DMA,
            pltpu.SemaphoreType.DMA,
        ],
    )(_sc_expand_body)
    return f(lc_flat, mapping)


def kernel(hidden_states, codebook_C, mapping_M):
    hs2d = hidden_states.reshape(R, H)
    lc = _codebook_logits(hs2d, codebook_C)
    m32 = mapping_M.astype(jnp.int32)
    out_flat = _expand_vocab(lc.reshape(R * K), m32)
    return _retile(out_flat)
